# Initial kernel scaffold; baseline (speedup 1.0000x reference)
import jax
import jax.numpy as jnp
from jax.experimental import pallas as pl


def kernel(seismic_features, audio_features, index, memory_l, memory_ab):
    raise NotImplementedError("write your pallas kernel here")


# speedup vs baseline: 7.4592x; 7.4592x over previous
import functools

import jax
import jax.numpy as jnp
from jax import lax
from jax.experimental import pallas as pl
from jax.experimental.pallas import tpu as pltpu
from jax.experimental.pallas import tpu_sc as plsc

_N = 100000
_B = 256
_D = 256
_KP1 = 512
_T = 0.07
_EPS = 1e-07

_NC = 2
_NS = 16
_L = 16


def _rne_bf16(x):
    u = lax.bitcast_convert_type(x, jnp.uint32)
    r = lax.shift_right_logical(u, jnp.uint32(16)) & jnp.uint32(1)
    u = (u + jnp.uint32(0x7FFF) + r) & jnp.uint32(0xFFFF0000)
    return lax.bitcast_convert_type(u, jnp.float32)


def _sample_idx(index):
    idx_key = jax.random.fold_in(jax.random.key(0), 1)
    idx = jax.random.randint(idx_key, (_B, _KP1), 0, _N).astype(jnp.int32)
    return idx.at[:, 0].set(index.astype(jnp.int32))


@functools.cache
def _build_sc(n, b, d, kp1, c_items, nc, ns):
    nw = nc * ns
    bpw = b // nw
    ipw = bpw * kp1
    cpb = kp1 // c_items
    nch = ipw // c_items
    grp = c_items // _L
    jmax = d // _L

    def body(mem_l, mem_ab, idx2d, af_hbm, sf_hbm, out_l, out_ab,
             idx_v, fa_v, fs_v, rows_l, rows_ab, res_l, res_ab, sem0, sem1):
        wid = lax.axis_index("s") * nc + lax.axis_index("c")

        pltpu.sync_copy(idx2d.at[pl.ds(wid * nch, nch)], idx_v)
        pltpu.sync_copy(af_hbm.at[pl.ds(wid * bpw * d, bpw * d)], fa_v)
        pltpu.sync_copy(sf_hbm.at[pl.ds(wid * bpw * d, bpw * d)], fs_v)

        sems = (sem0, sem1)

        def _idx(c):
            return idx_v.at[c]

        def _issue(c, s):
            iv = _idx(c)
            pltpu.async_copy(mem_l.at[iv], rows_l.at[s], sems[s])
            pltpu.async_copy(mem_ab.at[iv], rows_ab.at[s], sems[s])

        def _wait(c, s):
            iv = _idx(c)
            pltpu.make_async_copy(mem_l.at[iv], rows_l.at[s], sems[s]).wait()
            pltpu.make_async_copy(mem_ab.at[iv], rows_ab.at[s], sems[s]).wait()

        def _dot_pass(rows_s, feat_v, fbase, gbase, out_res, obase):
            fvecs = [_rne_bf16(feat_v[pl.ds(fbase + jv * _L, _L)])
                     for jv in range(jmax)]
            vecs = []
            for t in range(_L):
                accs = [jnp.zeros((_L,), jnp.float32) for _ in range(4)]
                for jv in range(jmax):
                    rv = _rne_bf16(rows_s[gbase + t, pl.ds(jv * _L, _L)])
                    accs[jv % 4] = accs[jv % 4] + rv * fvecs[jv]
                vecs.append((accs[0] + accs[1]) + (accs[2] + accs[3]))
            io = lax.iota(jnp.int32, _L)
            for sv in (8, 4, 2, 1):
                idxv = jnp.bitwise_xor(io, sv)
                mask = (io & sv) == 0
                half = len(vecs) // 2
                vecs = [
                    jnp.where(mask,
                              u + jnp.take_along_axis(u, idxv, axis=0),
                              v + jnp.take_along_axis(v, idxv, axis=0))
                    for u, v in zip(vecs[:half], vecs[half:])
                ]
            out_res[pl.ds(obase, _L)] = vecs[0]

        def _chunk(c, s):
            _wait(c, s)
            fbase = (c // cpb) * d

            def _group(g, carry):
                gbase = g * _L
                obase = c * c_items + g * _L
                _dot_pass(rows_ab.at[s], fs_v, fbase, gbase, res_l, obase)
                _dot_pass(rows_l.at[s], fa_v, fbase, gbase, res_ab, obase)
                return carry

            lax.fori_loop(0, grp, _group, 0)

            @pl.when(c + 2 < nch)
            def _():
                _issue(c + 2, s)

        _issue(0, 0)
        _issue(1, 1)

        def _outer(cc, carry):
            _chunk(cc * 2, 0)
            _chunk(cc * 2 + 1, 1)
            return carry

        lax.fori_loop(0, nch // 2, _outer, 0)

        pltpu.sync_copy(res_l, out_l.at[pl.ds(wid * ipw, ipw)])
        pltpu.sync_copy(res_ab, out_ab.at[pl.ds(wid * ipw, ipw)])

    mesh = plsc.VectorSubcoreMesh(core_axis_name="c", subcore_axis_name="s",
                                  num_cores=nc, num_subcores=ns)
    return pl.kernel(
        body,
        out_type=(jax.ShapeDtypeStruct((b * kp1,), jnp.float32),
                  jax.ShapeDtypeStruct((b * kp1,), jnp.float32)),
        mesh=mesh,
        scratch_types=[
            pltpu.VMEM((nch, c_items), jnp.int32),
            pltpu.VMEM((bpw * d,), jnp.float32),
            pltpu.VMEM((bpw * d,), jnp.float32),
            pltpu.VMEM((2, c_items, d), jnp.float32),
            pltpu.VMEM((2, c_items, d), jnp.float32),
            pltpu.VMEM((ipw,), jnp.float32),
            pltpu.VMEM((ipw,), jnp.float32),
            pltpu.SemaphoreType.DMA,
            pltpu.SemaphoreType.DMA,
        ],
        compiler_params=pltpu.CompilerParams(use_tc_tiling_on_sc=False),
    )


def _loss_body(ol_ref, oab_ref, out_ref):
    m_pn = float(_KP1 - 1) / float(_N)

    def half(o):
        s = jnp.exp(o * (1.0 / _T))
        z = jnp.mean(s) * _N
        p = s / z
        log_d1 = jnp.log(p / (p + m_pn + _EPS))
        log_d0 = jnp.log(m_pn / (p + m_pn + _EPS))
        col0 = lax.broadcasted_iota(jnp.int32, o.shape, 1) == 0
        contrib = jnp.where(col0, log_d1, log_d0)
        return -jnp.sum(contrib) / o.shape[0]

    out_ref[0, 0] = half(ol_ref[...]) + half(oab_ref[...])


def _loss_call(ol, oab):
    return pl.pallas_call(
        _loss_body,
        out_shape=jax.ShapeDtypeStruct((1, 1), jnp.float32),
        in_specs=[pl.BlockSpec(memory_space=pltpu.VMEM),
                  pl.BlockSpec(memory_space=pltpu.VMEM)],
        out_specs=pl.BlockSpec(memory_space=pltpu.SMEM),
    )(ol, oab)


def kernel(seismic_features, audio_features, index, memory_l, memory_ab):
    idx = _sample_idx(index)
    c_items = 64
    sc = _build_sc(_N, _B, _D, _KP1, c_items, _NC, _NS)
    out_l, out_ab = sc(
        memory_l, memory_ab,
        idx.reshape(-1, c_items),
        audio_features.reshape(-1),
        seismic_features.reshape(-1),
    )
    loss = _loss_call(out_l.reshape(_B, _KP1), out_ab.reshape(_B, _KP1))
    return loss.reshape(1)
